# baseline (speedup 1.0000x reference)
import functools

import numpy as np
import jax
import jax.numpy as jnp
from jax import lax
from jax.experimental import pallas as pl
from jax.experimental.pallas import tpu as pltpu
from jax.experimental.pallas import tpu_sc as plsc

LANES = 128
PHASES = 4
SHIFT = 32
GROUPS_PER_STEP = 8
NBUF = 2


def _gather_sc(emb_table, idx):
    B = idx.shape[0]
    _, D = emb_table.shape
    info = plsc.get_sparse_core_info()
    nw = info.num_cores * info.num_subcores
    b_per_w = B // nw
    mesh = plsc.VectorSubcoreMesh(core_axis_name="c", subcore_axis_name="s")

    @functools.partial(
        pl.kernel,
        mesh=mesh,
        out_type=jax.ShapeDtypeStruct((B, D), jnp.float32),
        scratch_types=[
            pltpu.VMEM((b_per_w,), jnp.int32),
            pltpu.VMEM((b_per_w, D), jnp.float32),
            pltpu.SemaphoreType.DMA,
        ],
        compiler_params=pltpu.CompilerParams(use_tc_tiling_on_sc=False),
    )
    def gather_kernel(table_hbm, idx_hbm, out_hbm, idx_v, rows_v, sem):
        wid = lax.axis_index("s") * info.num_cores + lax.axis_index("c")
        base = wid * b_per_w
        pltpu.sync_copy(idx_hbm.at[pl.ds(base, b_per_w)], idx_v)
        pltpu.async_copy(table_hbm.at[idx_v], rows_v, sem).wait()
        pltpu.sync_copy(rows_v, out_hbm.at[pl.ds(base, b_per_w)])

    return gather_kernel(emb_table, idx)


def _matmul_body(V, Vpad, x_ref, wt_hbm, b_hbm, out_ref, wt_v, b_v, sem):
    j = pl.program_id(0)
    G = GROUPS_PER_STEP
    B = x_ref.shape[0]
    mper = B // PHASES
    body = (V // LANES) * LANES
    m0 = pl.multiple_of(j * G, G)

    @pl.when(j == 0)
    def _load_weights():
        pltpu.make_async_copy(wt_hbm, wt_v, sem).start()
        pltpu.make_async_copy(wt_hbm, wt_v, sem).wait()
        pltpu.make_async_copy(b_hbm, b_v, sem).start()
        pltpu.make_async_copy(b_hbm, b_v, sem).wait()

    accs = []
    for r in range(PHASES):
        xblk = x_ref[pl.ds(mper * r + m0, G), :]
        accs.append(
            lax.dot_general(
                xblk,
                wt_v[r],
                (((1,), (0,)), ((), ())),
                preferred_element_type=jnp.float32,
            )
            + b_v[r]
        )

    out_ref[:, pl.ds(0, body)] = lax.slice(accs[0], (0, 0), (G, body))
    for r in range(1, PHASES):
        c = r * body
        prev_tail = lax.slice(accs[r - 1], (0, body), (G, body + LANES))
        head = lax.slice(accs[r], (0, 0), (G, LANES))
        mask = lax.broadcasted_iota(jnp.int32, (G, LANES), 1) < SHIFT * r
        out_ref[:, pl.ds(c, LANES)] = jnp.where(mask, prev_tail, head)
        width = body if r == PHASES - 1 else body - LANES
        out_ref[:, pl.ds(c + LANES, width)] = lax.slice(
            accs[r], (0, LANES), (G, LANES + width)
        )


def _project(x_perm, wt_shift, b_shift):
    B, D = x_perm.shape
    Vpad = wt_shift.shape[2]
    V = Vpad - LANES + SHIFT
    span = PHASES * V
    nm = B // PHASES
    nb = nm // GROUPS_PER_STEP
    return pl.pallas_call(
        functools.partial(_matmul_body, V, Vpad),
        grid=(nb,),
        in_specs=[
            pl.BlockSpec((B, D), lambda j: (0, 0)),
            pl.BlockSpec(memory_space=pl.ANY),
            pl.BlockSpec(memory_space=pl.ANY),
        ],
        out_specs=pl.BlockSpec((GROUPS_PER_STEP, span), lambda j: (j, 0)),
        out_shape=jax.ShapeDtypeStruct((nm, span), jnp.float32),
        scratch_shapes=[
            pltpu.VMEM((PHASES, D, Vpad), jnp.float32),
            pltpu.VMEM((PHASES, 1, Vpad), jnp.float32),
            pltpu.SemaphoreType.DMA,
        ],
        compiler_params=pltpu.CompilerParams(
            vmem_limit_bytes=100 * 1024 * 1024,
        ),
    )(x_perm, wt_shift, b_shift)


def kernel(inputs_, emb_table, lin_w, lin_b):
    B = inputs_.shape[0]
    V, D = lin_w.shape
    Vpad = ((V + SHIFT * (PHASES - 1)) + LANES - 1) // LANES * LANES

    perm = np.concatenate(
        [np.arange(r, B, PHASES) for r in range(PHASES)]
    ).astype(np.int32)
    idx_perm = inputs_.astype(jnp.int32)[perm]

    x_perm = _gather_sc(emb_table, idx_perm)

    wt = lin_w.T
    wt_shift = jnp.zeros((PHASES, D, Vpad), jnp.float32)
    b_shift = jnp.zeros((PHASES, 1, Vpad), jnp.float32)
    for r in range(PHASES):
        wt_shift = lax.dynamic_update_slice(wt_shift, wt[None], (r, 0, SHIFT * r))
        b_shift = lax.dynamic_update_slice(
            b_shift, lin_b[None, None], (r, 0, SHIFT * r)
        )

    out2 = _project(x_perm, wt_shift, b_shift)
    return out2.reshape(B, V)


# speedup vs baseline: 1.0006x; 1.0006x over previous
import functools

import numpy as np
import jax
import jax.numpy as jnp
from jax import lax
from jax.experimental import pallas as pl
from jax.experimental.pallas import tpu as pltpu
from jax.experimental.pallas import tpu_sc as plsc

LANES = 128
PHASES = 4
SHIFT = 32
GROUPS_PER_STEP = 8
NBUF = 2


def _gather_sc(emb_table, idx):
    B = idx.shape[0]
    _, D = emb_table.shape
    info = plsc.get_sparse_core_info()
    nw = info.num_cores * info.num_subcores
    b_per_w = B // nw
    mesh = plsc.VectorSubcoreMesh(core_axis_name="c", subcore_axis_name="s")

    @functools.partial(
        pl.kernel,
        mesh=mesh,
        out_type=jax.ShapeDtypeStruct((B, D), jnp.float32),
        scratch_types=[
            pltpu.VMEM((b_per_w,), jnp.int32),
            pltpu.VMEM((b_per_w, D), jnp.float32),
            pltpu.SemaphoreType.DMA,
        ],
        compiler_params=pltpu.CompilerParams(use_tc_tiling_on_sc=False),
    )
    def gather_kernel(table_hbm, idx_hbm, out_hbm, idx_v, rows_v, sem):
        wid = lax.axis_index("s") * info.num_cores + lax.axis_index("c")
        base = wid * b_per_w
        pltpu.sync_copy(idx_hbm.at[pl.ds(base, b_per_w)], idx_v)
        pltpu.async_copy(table_hbm.at[idx_v], rows_v, sem).wait()
        pltpu.sync_copy(rows_v, out_hbm.at[pl.ds(base, b_per_w)])

    return gather_kernel(emb_table, idx)


N_CHUNKS = 5


def _matmul_body(V, Vpad, x_ref, wt_hbm, b_hbm, out_hbm,
                 wt_v, b_v, wsem, *scratch):
    bufs = scratch[:NBUF]
    sems = scratch[NBUF:]
    j = pl.program_id(0)
    nb = pl.num_programs(0)
    slot = lax.rem(j, NBUF)
    G = GROUPS_PER_STEP
    B = x_ref.shape[0]
    mper = B // PHASES
    span = PHASES * V
    body = (V // LANES) * LANES
    chunk = span // N_CHUNKS
    m0 = pl.multiple_of(j * G, G)

    @pl.when(j == 0)
    def _load_weights():
        pltpu.make_async_copy(wt_hbm, wt_v, wsem).start()
        pltpu.make_async_copy(wt_hbm, wt_v, wsem).wait()
        pltpu.make_async_copy(b_hbm, b_v, wsem).start()
        pltpu.make_async_copy(b_hbm, b_v, wsem).wait()

    accs = []
    for r in range(PHASES):
        xblk = x_ref[pl.ds(mper * r + m0, G), :]
        accs.append(
            lax.dot_general(
                xblk,
                wt_v[r],
                (((1,), (0,)), ((), ())),
                preferred_element_type=jnp.float32,
            )
            + b_v[r]
        )

    def _assemble(s):
        buf = bufs[s]
        buf[:, pl.ds(0, body)] = lax.slice(accs[0], (0, 0), (G, body))
        for r in range(1, PHASES):
            c = r * body
            prev_tail = lax.slice(accs[r - 1], (0, body), (G, body + LANES))
            head = lax.slice(accs[r], (0, 0), (G, LANES))
            mask = lax.broadcasted_iota(jnp.int32, (G, LANES), 1) < SHIFT * r
            buf[:, pl.ds(c, LANES)] = jnp.where(mask, prev_tail, head)
            width = body if r == PHASES - 1 else body - LANES
            buf[:, pl.ds(c + LANES, width)] = lax.slice(
                accs[r], (0, LANES), (G, LANES + width)
            )

    for s in range(NBUF):
        @pl.when(jnp.logical_and(slot == s, j >= NBUF))
        def _wait_prev(s=s):
            pltpu.make_async_copy(
                bufs[s], out_hbm.at[pl.ds(0, G), :], sems[s]
            ).wait()

        @pl.when(slot == s)
        def _issue(s=s):
            _assemble(s)
            for c in range(N_CHUNKS):
                pltpu.make_async_copy(
                    bufs[s].at[:, pl.ds(c * chunk, chunk)],
                    out_hbm.at[pl.ds(m0, G), pl.ds(c * chunk, chunk)],
                    sems[s],
                ).start()

    @pl.when(j == nb - 1)
    def _drain():
        for s in range(NBUF):
            pltpu.make_async_copy(
                bufs[s], out_hbm.at[pl.ds(0, G), :], sems[s]
            ).wait()


def _project(x_perm, wt_shift, b_shift):
    B, D = x_perm.shape
    Vpad = wt_shift.shape[2]
    V = Vpad - LANES + SHIFT
    span = PHASES * V
    nm = B // PHASES
    nb = nm // GROUPS_PER_STEP
    return pl.pallas_call(
        functools.partial(_matmul_body, V, Vpad),
        grid=(nb,),
        in_specs=[
            pl.BlockSpec((B, D), lambda j: (0, 0)),
            pl.BlockSpec(memory_space=pl.ANY),
            pl.BlockSpec(memory_space=pl.ANY),
        ],
        out_specs=pl.BlockSpec(memory_space=pl.ANY),
        out_shape=jax.ShapeDtypeStruct((nm, span), jnp.float32),
        scratch_shapes=(
            [pltpu.VMEM((PHASES, D, Vpad), jnp.float32),
             pltpu.VMEM((PHASES, 1, Vpad), jnp.float32),
             pltpu.SemaphoreType.DMA]
            + [pltpu.VMEM((GROUPS_PER_STEP, span), jnp.float32)
               for _ in range(NBUF)]
            + [pltpu.SemaphoreType.DMA for _ in range(NBUF)]
        ),
        compiler_params=pltpu.CompilerParams(
            vmem_limit_bytes=100 * 1024 * 1024,
        ),
    )(x_perm, wt_shift, b_shift)


def kernel(inputs_, emb_table, lin_w, lin_b):
    B = inputs_.shape[0]
    V, D = lin_w.shape
    Vpad = ((V + SHIFT * (PHASES - 1)) + LANES - 1) // LANES * LANES

    perm = np.concatenate(
        [np.arange(r, B, PHASES) for r in range(PHASES)]
    ).astype(np.int32)
    idx_perm = inputs_.astype(jnp.int32)[perm]

    x_perm = _gather_sc(emb_table, idx_perm)

    wt = lin_w.T
    wt_shift = jnp.zeros((PHASES, D, Vpad), jnp.float32)
    b_shift = jnp.zeros((PHASES, 1, Vpad), jnp.float32)
    for r in range(PHASES):
        wt_shift = lax.dynamic_update_slice(wt_shift, wt[None], (r, 0, SHIFT * r))
        b_shift = lax.dynamic_update_slice(
            b_shift, lin_b[None, None], (r, 0, SHIFT * r)
        )

    out2 = _project(x_perm, wt_shift, b_shift)
    return out2.reshape(B, V)


# speedup vs baseline: 1.8606x; 1.8595x over previous
import functools

import jax
import jax.numpy as jnp
from jax import lax
from jax.experimental import pallas as pl
from jax.experimental.pallas import tpu as pltpu
from jax.experimental.pallas import tpu_sc as plsc

BATCH_BLOCK = 64


def _gather_sc(emb_table, idx):
    B = idx.shape[0]
    _, D = emb_table.shape
    info = plsc.get_sparse_core_info()
    nw = info.num_cores * info.num_subcores
    b_per_w = B // nw
    mesh = plsc.VectorSubcoreMesh(core_axis_name="c", subcore_axis_name="s")

    @functools.partial(
        pl.kernel,
        mesh=mesh,
        out_type=jax.ShapeDtypeStruct((B, D), jnp.float32),
        scratch_types=[
            pltpu.VMEM((b_per_w,), jnp.int32),
            pltpu.VMEM((b_per_w, D), jnp.float32),
            pltpu.SemaphoreType.DMA,
        ],
        compiler_params=pltpu.CompilerParams(use_tc_tiling_on_sc=False),
    )
    def gather_kernel(table_hbm, idx_hbm, out_hbm, idx_v, rows_v, sem):
        wid = lax.axis_index("s") * info.num_cores + lax.axis_index("c")
        base = wid * b_per_w
        pltpu.sync_copy(idx_hbm.at[pl.ds(base, b_per_w)], idx_v)
        pltpu.async_copy(table_hbm.at[idx_v], rows_v, sem).wait()
        pltpu.sync_copy(rows_v, out_hbm.at[pl.ds(base, b_per_w)])

    return gather_kernel(emb_table, idx)


def _matmul_body(x_ref, wt_ref, b_ref, out_ref):
    acc = lax.dot_general(
        x_ref[...],
        wt_ref[...],
        (((1,), (0,)), ((), ())),
        preferred_element_type=jnp.float32,
    )
    out_ref[...] = acc + b_ref[...]


def _project(x, lin_wt, lin_b2d):
    B, D = x.shape
    V = lin_wt.shape[1]
    nb = pl.cdiv(B, BATCH_BLOCK)
    return pl.pallas_call(
        _matmul_body,
        grid=(nb,),
        in_specs=[
            pl.BlockSpec((BATCH_BLOCK, D), lambda j: (j, 0)),
            pl.BlockSpec((D, V), lambda j: (0, 0)),
            pl.BlockSpec((1, V), lambda j: (0, 0)),
        ],
        out_specs=pl.BlockSpec((BATCH_BLOCK, V), lambda j: (j, 0)),
        out_shape=jax.ShapeDtypeStruct((B, V), jnp.float32),
        compiler_params=pltpu.CompilerParams(
            vmem_limit_bytes=100 * 1024 * 1024,
        ),
    )(x, lin_wt, lin_b2d)


def kernel(inputs_, emb_table, lin_w, lin_b):
    idx = inputs_.astype(jnp.int32)
    x = _gather_sc(emb_table, idx)
    return _project(x, lin_w.T, lin_b.reshape(1, -1))


# speedup vs baseline: 2.1130x; 1.1357x over previous
import functools

import jax
import jax.numpy as jnp
from jax import lax
from jax.experimental import pallas as pl
from jax.experimental.pallas import tpu as pltpu
from jax.experimental.pallas import tpu_sc as plsc

BATCH_BLOCK = 64


def _gather_sc(emb_table, idx):
    B = idx.shape[0]
    _, D = emb_table.shape
    info = plsc.get_sparse_core_info()
    nw = info.num_cores * info.num_subcores
    b_per_w = B // nw
    mesh = plsc.VectorSubcoreMesh(core_axis_name="c", subcore_axis_name="s")

    @functools.partial(
        pl.kernel,
        mesh=mesh,
        out_type=jax.ShapeDtypeStruct((B, D), jnp.float32),
        scratch_types=[
            pltpu.VMEM((b_per_w,), jnp.int32),
            pltpu.VMEM((b_per_w, D), jnp.float32),
            pltpu.SemaphoreType.DMA,
        ],
        compiler_params=pltpu.CompilerParams(use_tc_tiling_on_sc=False),
    )
    def gather_kernel(table_hbm, idx_hbm, out_hbm, idx_v, rows_v, sem):
        wid = lax.axis_index("s") * info.num_cores + lax.axis_index("c")
        base = wid * b_per_w
        pltpu.sync_copy(idx_hbm.at[pl.ds(base, b_per_w)], idx_v)
        pltpu.async_copy(table_hbm.at[idx_v], rows_v, sem).wait()
        pltpu.sync_copy(rows_v, out_hbm.at[pl.ds(base, b_per_w)])

    return gather_kernel(emb_table, idx)


def _matmul_body(x_ref, wt_ref, b_ref, out_ref):
    acc = lax.dot_general(
        x_ref[...],
        wt_ref[...],
        (((1,), (0,)), ((), ())),
        preferred_element_type=jnp.float32,
    )
    out_ref[...] = acc + b_ref[...]


def _project(x, lin_wt, lin_b2d):
    B, D = x.shape
    V = lin_wt.shape[1]
    nb = pl.cdiv(B, BATCH_BLOCK)
    return pl.pallas_call(
        _matmul_body,
        grid=(nb,),
        in_specs=[
            pl.BlockSpec((BATCH_BLOCK, D), lambda j: (j, 0)),
            pl.BlockSpec((D, V), lambda j: (0, 0)),
            pl.BlockSpec((1, V), lambda j: (0, 0)),
        ],
        out_specs=pl.BlockSpec((BATCH_BLOCK, V), lambda j: (j, 0)),
        out_shape=jax.ShapeDtypeStruct((B, V), jnp.float32),
        compiler_params=pltpu.CompilerParams(
            vmem_limit_bytes=100 * 1024 * 1024,
        ),
    )(x, lin_wt, lin_b2d)


def kernel(inputs_, emb_table, lin_w, lin_b):
    idx = inputs_.astype(jnp.int32)
    x = _gather_sc(emb_table, idx)
    V = lin_w.shape[0]
    vpad = (V + 127) // 128 * 128
    wt = jnp.pad(lin_w.T, ((0, 0), (0, vpad - V)))
    b2 = jnp.pad(lin_b.reshape(1, -1), ((0, 0), (0, vpad - V)))
    return _project(x, wt, b2)[:, :V]
